# baseline (speedup 1.0000x reference)
import functools

import jax
import jax.numpy as jnp
from jax import lax
from jax.experimental import pallas as pl
from jax.experimental.pallas import tpu as pltpu
from jax.experimental.pallas import tpu_sc as plsc

MAX_REL = 128
NC = 2
NS = 16
LANES = 16


def _sc_body(table_hbm, out_hbm, table_v, rows_v, full_v, shared, sem, *, S, D, B, T, NROWS):
    cid = lax.axis_index("c")
    sid = lax.axis_index("s")
    wid = cid * NS + sid
    NCH = D // LANES

    pltpu.sync_copy(table_hbm, table_v)

    i0 = jnp.minimum(sid * NROWS, S - NROWS)

    a = jnp.maximum(i0 - (S - 1 - MAX_REL), 0)
    bnd = jnp.minimum(i0 + MAX_REL, 2 * MAX_REL)

    def win_body(t, acc):
        return tuple(
            acc[c] + table_v[t, pl.ds(c * LANES, LANES)] for c in range(NCH)
        )

    acc = tuple(jnp.zeros((LANES,), jnp.float32) for _ in range(NCH))
    acc = lax.fori_loop(a, bnd + 1, win_body, acc)
    lo_f = jnp.maximum((S - 1 - MAX_REL) - i0, 0).astype(jnp.float32)
    hi_f = jnp.maximum(i0 - MAX_REL, 0).astype(jnp.float32)
    acc = tuple(
        acc[c]
        + lo_f * table_v[0, pl.ds(c * LANES, LANES)]
        + hi_f * table_v[2 * MAX_REL, pl.ds(c * LANES, LANES)]
        for c in range(NCH)
    )
    for c in range(NCH):
        rows_v[0, pl.ds(c * LANES, LANES)] = acc[c]

    for r in range(1, NROWS):
        i = i0 + r
        add_t = jnp.minimum(i + MAX_REL, 2 * MAX_REL)
        sub_t = jnp.maximum(i - (S - 1 - MAX_REL) - 1, 0)
        acc = tuple(
            acc[c]
            + table_v[add_t, pl.ds(c * LANES, LANES)]
            - table_v[sub_t, pl.ds(c * LANES, LANES)]
            for c in range(NCH)
        )
        for c in range(NCH):
            rows_v[r, pl.ds(c * LANES, LANES)] = acc[c]

    pltpu.sync_copy(rows_v, shared.at[pl.ds(i0, NROWS)])
    plsc.subcore_barrier()
    pltpu.sync_copy(shared, full_v)

    bpw = B // (NC * NS)
    base = wid * bpw
    K = 8
    for g in range(bpw // K):
        for k in range(K):
            pltpu.make_async_copy(
                full_v, out_hbm.at[base + g * K + k], sem
            ).start()
        for k in range(K):
            pltpu.make_async_copy(
                full_v, out_hbm.at[base + g * K + k], sem
            ).wait()


def _sc_broadcast(table, B, S, D):
    T = table.shape[0]
    NROWS = (S + NS - 1) // NS
    mesh = plsc.VectorSubcoreMesh(core_axis_name="c", subcore_axis_name="s")
    kfn = pl.kernel(
        functools.partial(_sc_body, S=S, D=D, B=B, T=T, NROWS=NROWS),
        mesh=mesh,
        out_type=jax.ShapeDtypeStruct((B, S, D), jnp.float32),
        scratch_types=[
            pltpu.VMEM((T, D), jnp.float32),
            pltpu.VMEM((NROWS, D), jnp.float32),
            pltpu.VMEM((S, D), jnp.float32),
            pltpu.VMEM_SHARED((S, D), jnp.float32),
            pltpu.SemaphoreType.DMA,
        ],
    )
    return kfn(table)


def kernel(inputs, table):
    B, S = inputs.shape
    T, D = table.shape
    return _sc_broadcast(table, B, S, D)
